# Initial kernel scaffold; baseline (speedup 1.0000x reference)
import jax
import jax.numpy as jnp
from jax.experimental import pallas as pl


def kernel(x, edge_index, W, b):
    raise NotImplementedError("write your pallas kernel here")


# speedup vs baseline: 20.3719x; 20.3719x over previous
import jax
import jax.numpy as jnp
from jax import lax
from jax.experimental import pallas as pl
from jax.experimental.pallas import tpu as pltpu
from jax.experimental.pallas import tpu_sc as plsc

N = 10000
IN_CH = 128
D = 64
E = 320000
NC = 2
NS = 16
NW = NC * NS
CH = 128
NCH = 80
EPT = CH * NCH
E_PAD = EPT * NW
N_PAD = 10240
RPT = N_PAD // NS

_mesh = plsc.VectorSubcoreMesh(
    core_axis_name="c", subcore_axis_name="s", num_cores=NC, num_subcores=NS
)
_sc_params = pltpu.CompilerParams(use_tc_tiling_on_sc=False)


def _deg_body(dst_hbm, ones_hbm, zeros_hbm, deg_out, idx_v, ones_v, deg_sh):
    c = lax.axis_index("c")
    s = lax.axis_index("s")
    wid = c * NS + s
    pltpu.sync_copy(dst_hbm.at[pl.ds(wid * NCH, NCH), :], idx_v)
    pltpu.sync_copy(ones_hbm, ones_v)
    pltpu.sync_copy(zeros_hbm.at[pl.ds(s * RPT, RPT)], deg_sh.at[pl.ds(s * RPT, RPT)])
    plsc.subcore_barrier()

    def body(j, carry):
        pltpu.sync_copy(ones_v, deg_sh.at[idx_v.at[j]], add=True)
        return carry

    lax.fori_loop(0, NCH, body, 0)
    plsc.subcore_barrier()
    pltpu.sync_copy(
        deg_sh.at[pl.ds(s * RPT, RPT)],
        deg_out.at[pl.ds(c * N_PAD + s * RPT, RPT)],
    )


_deg_kernel = pl.kernel(
    _deg_body,
    out_type=jax.ShapeDtypeStruct((NC * N_PAD,), jnp.float32),
    mesh=_mesh,
    scratch_types=[
        pltpu.VMEM((NCH, CH), jnp.int32),
        pltpu.VMEM((CH,), jnp.float32),
        pltpu.VMEM_SHARED((N_PAD,), jnp.float32),
    ],
    compiler_params=_sc_params,
)


def _scat_body(src_hbm, dst_hbm, g_hbm, zeros_hbm, acc_out,
               sidx_v, didx_v, gbuf, acc_sh, sem):
    c = lax.axis_index("c")
    s = lax.axis_index("s")
    wid = c * NS + s
    pltpu.sync_copy(src_hbm.at[pl.ds(wid * NCH, NCH), :], sidx_v)
    pltpu.sync_copy(dst_hbm.at[pl.ds(wid * NCH, NCH), :], didx_v)
    pltpu.sync_copy(
        zeros_hbm.at[pl.ds(s * RPT, RPT), :], acc_sh.at[pl.ds(s * RPT, RPT), :]
    )
    plsc.subcore_barrier()

    def body(j, carry):
        pltpu.async_copy(g_hbm.at[sidx_v.at[j]], gbuf, sem).wait()
        pltpu.sync_copy(gbuf, acc_sh.at[didx_v.at[j]], add=True)
        return carry

    lax.fori_loop(0, NCH, body, 0)
    plsc.subcore_barrier()
    pltpu.sync_copy(
        acc_sh.at[pl.ds(s * RPT, RPT), :],
        acc_out.at[pl.ds(c * N_PAD + s * RPT, RPT), :],
    )


_scat_kernel = pl.kernel(
    _scat_body,
    out_type=jax.ShapeDtypeStruct((NC * N_PAD, D), jnp.float32),
    mesh=_mesh,
    scratch_types=[
        pltpu.VMEM((NCH, CH), jnp.int32),
        pltpu.VMEM((NCH, CH), jnp.int32),
        pltpu.VMEM((CH, D), jnp.float32),
        pltpu.VMEM_SHARED((N_PAD, D), jnp.float32),
        pltpu.SemaphoreType.DMA,
    ],
    compiler_params=_sc_params,
)


def _proj_body(x_ref, w_ref, b_ref, deg_ref, g_ref):
    h = lax.dot_general(
        x_ref[...], w_ref[...], (((1,), (1,)), ((), ())),
        preferred_element_type=jnp.float32,
    )
    h = h + b_ref[...]
    nrm = jnp.sqrt(jnp.sum(h * h, axis=1, keepdims=True))
    h = (h / jnp.maximum(nrm, 1e-12)) * 1.8
    d = deg_ref[pl.ds(0, N), :] + deg_ref[pl.ds(N_PAD, N), :] + 1.0
    g_ref[...] = h * lax.rsqrt(d)


_proj = pl.pallas_call(
    _proj_body,
    out_shape=jax.ShapeDtypeStruct((N, D), jnp.float32),
)


def _out_body(g_ref, acc_ref, deg_ref, o_ref):
    d = deg_ref[pl.ds(0, N), :] + deg_ref[pl.ds(N_PAD, N), :] + 1.0
    o_ref[...] = lax.rsqrt(d) * (
        g_ref[...] + acc_ref[pl.ds(0, N), :] + acc_ref[pl.ds(N_PAD, N), :]
    )


_out = pl.pallas_call(
    _out_body,
    out_shape=jax.ShapeDtypeStruct((N, D), jnp.float32),
)


def kernel(x, edge_index, W, b):
    ei = edge_index.astype(jnp.int32)
    src = ei[0]
    dst = ei[1]
    pad = E_PAD - E
    src_p = jnp.concatenate([src, jnp.zeros((pad,), jnp.int32)])
    dst_p = jnp.concatenate([dst, jnp.full((pad,), N, jnp.int32)])
    src2d = src_p.reshape(NW * NCH, CH)
    dst2d = dst_p.reshape(NW * NCH, CH)
    ones = jnp.ones((CH,), jnp.float32)
    zeros1 = jnp.zeros((N_PAD,), jnp.float32)
    zeros2 = jnp.zeros((N_PAD, D), jnp.float32)

    degp = _deg_kernel(dst2d, ones, zeros1)
    degp2 = degp.reshape(NC * N_PAD, 1)
    g = _proj(x, W, b.reshape(1, D), degp2)
    accp = _scat_kernel(src2d, dst2d, g, zeros2)
    return _out(g, accp, degp2)


# speedup vs baseline: 23.7540x; 1.1660x over previous
import jax
import jax.numpy as jnp
from jax import lax
from jax.experimental import pallas as pl
from jax.experimental.pallas import tpu as pltpu
from jax.experimental.pallas import tpu_sc as plsc

N = 10000
IN_CH = 128
D = 64
E = 320000
NC = 2
NS = 16
NW = NC * NS
CH = 128
NCH = 80
EPT = CH * NCH
E_PAD = EPT * NW
N_PAD = 10240
RPT = N_PAD // NS

_mesh = plsc.VectorSubcoreMesh(
    core_axis_name="c", subcore_axis_name="s", num_cores=NC, num_subcores=NS
)
_sc_params = pltpu.CompilerParams(use_tc_tiling_on_sc=False)


DEPTH = 8


def _deg_body(dst_hbm, ones_hbm, zeros_hbm, deg_out, idx_v, ones_v, deg_sh, dsem):
    c = lax.axis_index("c")
    s = lax.axis_index("s")
    wid = c * NS + s
    pltpu.sync_copy(dst_hbm.at[pl.ds(wid * NCH, NCH), :], idx_v)
    pltpu.sync_copy(ones_hbm, ones_v)
    pltpu.sync_copy(zeros_hbm.at[pl.ds(s * RPT, RPT)], deg_sh.at[pl.ds(s * RPT, RPT)])
    plsc.subcore_barrier()

    def body(i, carry):
        for b in range(DEPTH):
            pltpu.async_copy(ones_v, deg_sh.at[idx_v.at[i * DEPTH + b]], dsem,
                             add=True)
        for b in range(DEPTH):
            pltpu.make_async_copy(
                ones_v, deg_sh.at[idx_v.at[i * DEPTH + b]], dsem
            ).wait()
        return carry

    lax.fori_loop(0, NCH // DEPTH, body, 0)
    plsc.subcore_barrier()
    pltpu.sync_copy(
        deg_sh.at[pl.ds(s * RPT, RPT)],
        deg_out.at[pl.ds(c * N_PAD + s * RPT, RPT)],
    )


_deg_kernel = pl.kernel(
    _deg_body,
    out_type=jax.ShapeDtypeStruct((NC * N_PAD,), jnp.float32),
    mesh=_mesh,
    scratch_types=[
        pltpu.VMEM((NCH, CH), jnp.int32),
        pltpu.VMEM((CH,), jnp.float32),
        pltpu.VMEM_SHARED((N_PAD,), jnp.float32),
        pltpu.SemaphoreType.DMA,
    ],
    compiler_params=_sc_params,
)


def _scat_body(src_hbm, dst_hbm, g_hbm, zeros_hbm, acc_out,
               sidx_v, didx_v, gbuf, acc_sh, sem):
    c = lax.axis_index("c")
    s = lax.axis_index("s")
    wid = c * NS + s
    pltpu.sync_copy(src_hbm.at[pl.ds(wid * NCH, NCH), :], sidx_v)
    pltpu.sync_copy(dst_hbm.at[pl.ds(wid * NCH, NCH), :], didx_v)
    pltpu.sync_copy(
        zeros_hbm.at[pl.ds(s * RPT, RPT), :], acc_sh.at[pl.ds(s * RPT, RPT), :]
    )
    plsc.subcore_barrier()
    for b in range(DEPTH):
        pltpu.async_copy(g_hbm.at[sidx_v.at[b]], gbuf.at[b], sem.at[b])

    def body(i, carry):
        for b in range(DEPTH):
            j = i * DEPTH + b
            pltpu.make_async_copy(
                g_hbm.at[sidx_v.at[j]], gbuf.at[b], sem.at[b]
            ).wait()
            pltpu.sync_copy(gbuf.at[b], acc_sh.at[didx_v.at[j]], add=True)

            @pl.when(j + DEPTH < NCH)
            def _():
                pltpu.async_copy(
                    g_hbm.at[sidx_v.at[j + DEPTH]], gbuf.at[b], sem.at[b]
                )

        return carry

    lax.fori_loop(0, NCH // DEPTH, body, 0)
    plsc.subcore_barrier()
    pltpu.sync_copy(
        acc_sh.at[pl.ds(s * RPT, RPT), :],
        acc_out.at[pl.ds(c * N_PAD + s * RPT, RPT), :],
    )


_scat_kernel = pl.kernel(
    _scat_body,
    out_type=jax.ShapeDtypeStruct((NC * N_PAD, D), jnp.float32),
    mesh=_mesh,
    scratch_types=[
        pltpu.VMEM((NCH, CH), jnp.int32),
        pltpu.VMEM((NCH, CH), jnp.int32),
        pltpu.VMEM((DEPTH, CH, D), jnp.float32),
        pltpu.VMEM_SHARED((N_PAD, D), jnp.float32),
        pltpu.SemaphoreType.DMA((DEPTH,)),
    ],
    compiler_params=_sc_params,
)


def _proj_body(x_ref, w_ref, b_ref, deg_ref, g_ref):
    h = lax.dot_general(
        x_ref[...], w_ref[...], (((1,), (1,)), ((), ())),
        preferred_element_type=jnp.float32,
    )
    h = h + b_ref[...]
    nrm = jnp.sqrt(jnp.sum(h * h, axis=1, keepdims=True))
    h = (h / jnp.maximum(nrm, 1e-12)) * 1.8
    d = deg_ref[pl.ds(0, N), :] + deg_ref[pl.ds(N_PAD, N), :] + 1.0
    g_ref[...] = h * lax.rsqrt(d)


_proj = pl.pallas_call(
    _proj_body,
    out_shape=jax.ShapeDtypeStruct((N, D), jnp.float32),
)


def _out_body(g_ref, acc_ref, deg_ref, o_ref):
    d = deg_ref[pl.ds(0, N), :] + deg_ref[pl.ds(N_PAD, N), :] + 1.0
    o_ref[...] = lax.rsqrt(d) * (
        g_ref[...] + acc_ref[pl.ds(0, N), :] + acc_ref[pl.ds(N_PAD, N), :]
    )


_out = pl.pallas_call(
    _out_body,
    out_shape=jax.ShapeDtypeStruct((N, D), jnp.float32),
)


def kernel(x, edge_index, W, b):
    ei = edge_index.astype(jnp.int32)
    src = ei[0]
    dst = ei[1]
    pad = E_PAD - E
    src_p = jnp.concatenate([src, jnp.zeros((pad,), jnp.int32)])
    dst_p = jnp.concatenate([dst, jnp.full((pad,), N, jnp.int32)])
    src2d = src_p.reshape(NW * NCH, CH)
    dst2d = dst_p.reshape(NW * NCH, CH)
    ones = jnp.ones((CH,), jnp.float32)
    zeros1 = jnp.zeros((N_PAD,), jnp.float32)
    zeros2 = jnp.zeros((N_PAD, D), jnp.float32)

    degp = _deg_kernel(dst2d, ones, zeros1)
    degp2 = degp.reshape(NC * N_PAD, 1)
    g = _proj(x, W, b.reshape(1, D), degp2)
    accp = _scat_kernel(src2d, dst2d, g, zeros2)
    return _out(g, accp, degp2)


# speedup vs baseline: 41.5981x; 1.7512x over previous
import jax
import jax.numpy as jnp
from jax import lax
from jax.experimental import pallas as pl
from jax.experimental.pallas import tpu as pltpu
from jax.experimental.pallas import tpu_sc as plsc

N = 10000
IN_CH = 128
D = 64
HD = D // 2
E = 320000
NC = 2
NS = 16
NW = NC * NS
CH = 128
NCHD = 80
NCH = 160
E_PAD = CH * NCHD * NW
N_PAD = 10240
RPT = N_PAD // NS
DEPTH = 8

_mesh = plsc.VectorSubcoreMesh(
    core_axis_name="c", subcore_axis_name="s", num_cores=NC, num_subcores=NS
)
_sc_params = pltpu.CompilerParams(use_tc_tiling_on_sc=False)


def _deg_body(dst_hbm, ones_hbm, zeros_hbm, deg_out, idx_v, ones_v, deg_sh, dsem):
    c = lax.axis_index("c")
    s = lax.axis_index("s")
    wid = c * NS + s
    pltpu.sync_copy(dst_hbm.at[pl.ds(wid * NCHD, NCHD), :], idx_v)
    pltpu.sync_copy(ones_hbm, ones_v)
    for q in range(RPT // CH):
        pltpu.sync_copy(zeros_hbm, deg_sh.at[pl.ds(s * RPT + q * CH, CH)])
    plsc.subcore_barrier()

    def body(i, carry):
        for b in range(DEPTH):
            pltpu.async_copy(ones_v, deg_sh.at[idx_v.at[i * DEPTH + b]], dsem,
                             add=True)
        for b in range(DEPTH):
            pltpu.make_async_copy(
                ones_v, deg_sh.at[idx_v.at[i * DEPTH + b]], dsem
            ).wait()
        return carry

    lax.fori_loop(0, NCHD // DEPTH, body, 0)
    plsc.subcore_barrier()
    pltpu.sync_copy(
        deg_sh.at[pl.ds(s * RPT, RPT)],
        deg_out.at[pl.ds(c * N_PAD + s * RPT, RPT)],
    )


_deg_kernel = pl.kernel(
    _deg_body,
    out_type=jax.ShapeDtypeStruct((NC * N_PAD,), jnp.float32),
    mesh=_mesh,
    scratch_types=[
        pltpu.VMEM((NCHD, CH), jnp.int32),
        pltpu.VMEM((CH,), jnp.float32),
        pltpu.VMEM_SHARED((N_PAD,), jnp.float32),
        pltpu.SemaphoreType.DMA,
    ],
    compiler_params=_sc_params,
)


def _scat_body(src_hbm, dst_hbm, g0_hbm, g1_hbm, zeros_hbm, acc_out,
               sidx_v, didx_v, gbuf, g_sh, acc_sh, sem):
    c = lax.axis_index("c")
    s = lax.axis_index("s")
    pltpu.sync_copy(src_hbm.at[pl.ds(s * NCH, NCH), :], sidx_v)
    pltpu.sync_copy(dst_hbm.at[pl.ds(s * NCH, NCH), :], didx_v)
    for q in range(RPT // CH):
        pltpu.sync_copy(zeros_hbm, acc_sh.at[pl.ds(s * RPT + q * CH, CH), :])
    base = s * RPT

    @pl.when(c == 0)
    def _():
        pltpu.sync_copy(g0_hbm.at[pl.ds(base, RPT), :], g_sh.at[pl.ds(base, RPT), :])

    @pl.when(c == 1)
    def _():
        pltpu.sync_copy(g1_hbm.at[pl.ds(base, RPT), :], g_sh.at[pl.ds(base, RPT), :])

    plsc.subcore_barrier()
    for b in range(DEPTH):
        pltpu.async_copy(g_sh.at[sidx_v.at[b]], gbuf.at[b], sem.at[b])

    def body(i, carry):
        for b in range(DEPTH):
            j = i * DEPTH + b
            pltpu.make_async_copy(
                g_sh.at[sidx_v.at[j]], gbuf.at[b], sem.at[b]
            ).wait()
            pltpu.sync_copy(gbuf.at[b], acc_sh.at[didx_v.at[j]], add=True)

            @pl.when(j + DEPTH < NCH)
            def _():
                pltpu.async_copy(
                    g_sh.at[sidx_v.at[j + DEPTH]], gbuf.at[b], sem.at[b]
                )

        return carry

    lax.fori_loop(0, NCH // DEPTH, body, 0)
    plsc.subcore_barrier()
    pltpu.sync_copy(
        acc_sh.at[pl.ds(s * RPT, RPT), :],
        acc_out.at[pl.ds(s * RPT, RPT), pl.ds(c * HD, HD)],
    )


_scat_kernel = pl.kernel(
    _scat_body,
    out_type=jax.ShapeDtypeStruct((N_PAD, D), jnp.float32),
    mesh=_mesh,
    scratch_types=[
        pltpu.VMEM((NCH, CH), jnp.int32),
        pltpu.VMEM((NCH, CH), jnp.int32),
        pltpu.VMEM((DEPTH, CH, HD), jnp.float32),
        pltpu.VMEM_SHARED((N_PAD, HD), jnp.float32),
        pltpu.VMEM_SHARED((N_PAD, HD), jnp.float32),
        pltpu.SemaphoreType.DMA((DEPTH,)),
    ],
    compiler_params=_sc_params,
)


def _proj_body(x_ref, w_ref, b_ref, deg_ref, g0_ref, g1_ref):
    h = lax.dot_general(
        x_ref[...], w_ref[...], (((1,), (1,)), ((), ())),
        preferred_element_type=jnp.float32,
    )
    h = h + b_ref[...]
    nrm = jnp.sqrt(jnp.sum(h * h, axis=1, keepdims=True))
    h = (h / jnp.maximum(nrm, 1e-12)) * 1.8
    d = deg_ref[pl.ds(0, N), :] + deg_ref[pl.ds(N_PAD, N), :] + 1.0
    g = h * lax.rsqrt(d)
    g0_ref[pl.ds(0, N), :] = g[:, :HD]
    g1_ref[pl.ds(0, N), :] = g[:, HD:]
    tail = jnp.zeros((N_PAD - N, HD), jnp.float32)
    g0_ref[pl.ds(N, N_PAD - N), :] = tail
    g1_ref[pl.ds(N, N_PAD - N), :] = tail


_proj = pl.pallas_call(
    _proj_body,
    out_shape=(
        jax.ShapeDtypeStruct((N_PAD, HD), jnp.float32),
        jax.ShapeDtypeStruct((N_PAD, HD), jnp.float32),
    ),
)


def _out_body(g0_ref, g1_ref, acc_ref, deg_ref, o_ref):
    d = deg_ref[pl.ds(0, N), :] + deg_ref[pl.ds(N_PAD, N), :] + 1.0
    g = jnp.concatenate([g0_ref[pl.ds(0, N), :], g1_ref[pl.ds(0, N), :]], axis=1)
    o_ref[...] = lax.rsqrt(d) * (g + acc_ref[pl.ds(0, N), :])


_out = pl.pallas_call(
    _out_body,
    out_shape=jax.ShapeDtypeStruct((N, D), jnp.float32),
)


def kernel(x, edge_index, W, b):
    ei = edge_index.astype(jnp.int32)
    src = ei[0]
    dst = ei[1]
    pad = E_PAD - E
    src_p = jnp.concatenate([src, jnp.zeros((pad,), jnp.int32)])
    dst_p = jnp.concatenate([dst, jnp.full((pad,), N, jnp.int32)])
    src2d = src_p.reshape(NS * NCH, CH)
    dst2d = dst_p.reshape(NS * NCH, CH)
    ones = jnp.ones((CH,), jnp.float32)
    zeros1 = jnp.zeros((CH,), jnp.float32)
    zeros2 = jnp.zeros((CH, HD), jnp.float32)

    degp = _deg_kernel(dst2d, ones, zeros1)
    degp2 = degp.reshape(NC * N_PAD, 1)
    g0, g1 = _proj(x, W, b.reshape(1, D), degp2)
    acc = _scat_kernel(src2d, dst2d, g0, g1, zeros2)
    return _out(g0, g1, acc, degp2)


# speedup vs baseline: 46.8157x; 1.1254x over previous
import jax
import jax.numpy as jnp
from jax import lax
from jax.experimental import pallas as pl
from jax.experimental.pallas import tpu as pltpu
from jax.experimental.pallas import tpu_sc as plsc

N = 10000
IN_CH = 128
D = 64
HD = D // 2
E = 320000
NC = 2
NS = 16
NW = NC * NS
CH = 128
NCHUNK = E // CH
DEG_Q, DEG_R = divmod(NCHUNK, NW)
SC_Q, SC_R = divmod(NCHUNK, NS)
N_PAD = 10240
RPT = N_PAD // NS
DEPTH = 8
WIN = 16

_mesh = plsc.VectorSubcoreMesh(
    core_axis_name="c", subcore_axis_name="s", num_cores=NC, num_subcores=NS
)
_sc_params = pltpu.CompilerParams(use_tc_tiling_on_sc=False)


def _deg_body(e_hbm, ones_hbm, zeros_hbm, deg_out, idx_v, ones_v, deg_sh, dsem):
    c = lax.axis_index("c")
    s = lax.axis_index("s")
    wid = c * NS + s
    cnt = DEG_Q + jnp.where(wid < DEG_R, 1, 0)
    lo = NCHUNK + wid * DEG_Q + jnp.minimum(wid, DEG_R)
    pltpu.sync_copy(e_hbm.at[pl.ds(lo, DEG_Q), :], idx_v.at[pl.ds(0, DEG_Q), :])

    @pl.when(wid < DEG_R)
    def _():
        pltpu.sync_copy(e_hbm.at[pl.ds(lo + DEG_Q, 1), :],
                        idx_v.at[pl.ds(DEG_Q, 1), :])

    pltpu.sync_copy(ones_hbm, ones_v)
    for q in range(RPT // CH):
        pltpu.sync_copy(zeros_hbm, deg_sh.at[pl.ds(s * RPT + q * CH, CH)])
    plsc.subcore_barrier()

    def body(i, carry):
        for b in range(DEPTH):
            pltpu.async_copy(ones_v, deg_sh.at[idx_v.at[i * DEPTH + b]], dsem,
                             add=True)
        for b in range(DEPTH):
            pltpu.make_async_copy(
                ones_v, deg_sh.at[idx_v.at[i * DEPTH + b]], dsem
            ).wait()
        return carry

    lax.fori_loop(0, DEG_Q // DEPTH, body, 0)

    def tail(j, carry):
        pltpu.sync_copy(ones_v, deg_sh.at[idx_v.at[j]], add=True)
        return carry

    lax.fori_loop((DEG_Q // DEPTH) * DEPTH, cnt, tail, 0)
    plsc.subcore_barrier()
    pltpu.sync_copy(
        deg_sh.at[pl.ds(s * RPT, RPT)],
        deg_out.at[pl.ds(c * N_PAD + s * RPT, RPT)],
    )


_deg_kernel = pl.kernel(
    _deg_body,
    out_type=jax.ShapeDtypeStruct((NC * N_PAD,), jnp.float32),
    mesh=_mesh,
    scratch_types=[
        pltpu.VMEM((DEG_Q + 1, CH), jnp.int32),
        pltpu.VMEM((CH,), jnp.float32),
        pltpu.VMEM_SHARED((N_PAD,), jnp.float32),
        pltpu.SemaphoreType.DMA,
    ],
    compiler_params=_sc_params,
)

_ITER = -(-(SC_Q + 1) // DEPTH)


def _scat_body(e_hbm, g0_hbm, g1_hbm, zeros_hbm, acc_out,
               sidx_v, didx_v, gbuf, g_sh, acc_sh, gsem):
    c = lax.axis_index("c")
    s = lax.axis_index("s")
    cnt = SC_Q + jnp.where(s < SC_R, 1, 0)
    lo = s * SC_Q + jnp.minimum(s, SC_R)
    pltpu.sync_copy(e_hbm.at[pl.ds(lo, SC_Q), :], sidx_v.at[pl.ds(0, SC_Q), :])
    pltpu.sync_copy(e_hbm.at[pl.ds(NCHUNK + lo, SC_Q), :],
                    didx_v.at[pl.ds(0, SC_Q), :])

    @pl.when(s < SC_R)
    def _():
        pltpu.sync_copy(e_hbm.at[pl.ds(lo + SC_Q, 1), :],
                        sidx_v.at[pl.ds(SC_Q, 1), :])
        pltpu.sync_copy(e_hbm.at[pl.ds(NCHUNK + lo + SC_Q, 1), :],
                        didx_v.at[pl.ds(SC_Q, 1), :])

    for q in range(RPT // CH):
        pltpu.sync_copy(zeros_hbm, acc_sh.at[pl.ds(s * RPT + q * CH, CH), :])
    base = s * RPT

    @pl.when(c == 0)
    def _():
        pltpu.sync_copy(g0_hbm.at[pl.ds(base, RPT), :], g_sh.at[pl.ds(base, RPT), :])

    @pl.when(c == 1)
    def _():
        pltpu.sync_copy(g1_hbm.at[pl.ds(base, RPT), :], g_sh.at[pl.ds(base, RPT), :])

    plsc.subcore_barrier()
    for b in range(DEPTH):
        pltpu.async_copy(g_sh.at[sidx_v.at[b]], gbuf.at[b], gsem.at[b])

    def body(i, carry):
        j0 = i * DEPTH
        for b in range(DEPTH):
            j = j0 + b

            @pl.when(j < cnt)
            def _():
                pltpu.make_async_copy(
                    g_sh.at[sidx_v.at[j]], gbuf.at[b], gsem.at[b]
                ).wait()
                pltpu.sync_copy(gbuf.at[b], acc_sh.at[didx_v.at[j]], add=True)

            jn = j + DEPTH

            @pl.when(jn < cnt)
            def _():
                pltpu.async_copy(g_sh.at[sidx_v.at[jn]], gbuf.at[b], gsem.at[b])

        return carry

    lax.fori_loop(0, _ITER, body, 0)
    plsc.subcore_barrier()
    pltpu.sync_copy(
        acc_sh.at[pl.ds(s * RPT, RPT), :],
        acc_out.at[pl.ds(s * RPT, RPT), pl.ds(c * HD, HD)],
    )


_scat_kernel = pl.kernel(
    _scat_body,
    out_type=jax.ShapeDtypeStruct((N_PAD, D), jnp.float32),
    mesh=_mesh,
    scratch_types=[
        pltpu.VMEM((SC_Q + 1, CH), jnp.int32),
        pltpu.VMEM((SC_Q + 1, CH), jnp.int32),
        pltpu.VMEM((DEPTH, CH, HD), jnp.float32),
        pltpu.VMEM_SHARED((N_PAD, HD), jnp.float32),
        pltpu.VMEM_SHARED((N_PAD, HD), jnp.float32),
        pltpu.SemaphoreType.DMA((DEPTH,)),
    ],
    compiler_params=_sc_params,
)


def _proj_body(x_ref, w_ref, b_ref, deg_ref, g0_ref, g1_ref):
    h = lax.dot_general(
        x_ref[...], w_ref[...], (((1,), (1,)), ((), ())),
        preferred_element_type=jnp.float32,
    )
    h = h + b_ref[...]
    nrm = jnp.sqrt(jnp.sum(h * h, axis=1, keepdims=True))
    h = (h / jnp.maximum(nrm, 1e-12)) * 1.8
    d = deg_ref[pl.ds(0, N), :] + deg_ref[pl.ds(N_PAD, N), :] + 1.0
    g = h * lax.rsqrt(d)
    g0_ref[pl.ds(0, N), :] = g[:, :HD]
    g1_ref[pl.ds(0, N), :] = g[:, HD:]
    tail = jnp.zeros((N_PAD - N, HD), jnp.float32)
    g0_ref[pl.ds(N, N_PAD - N), :] = tail
    g1_ref[pl.ds(N, N_PAD - N), :] = tail


_proj = pl.pallas_call(
    _proj_body,
    out_shape=(
        jax.ShapeDtypeStruct((N_PAD, HD), jnp.float32),
        jax.ShapeDtypeStruct((N_PAD, HD), jnp.float32),
    ),
)


def _out_body(g0_ref, g1_ref, acc_ref, deg_ref, o_ref):
    d = deg_ref[pl.ds(0, N), :] + deg_ref[pl.ds(N_PAD, N), :] + 1.0
    g = jnp.concatenate([g0_ref[pl.ds(0, N), :], g1_ref[pl.ds(0, N), :]], axis=1)
    o_ref[...] = lax.rsqrt(d) * (g + acc_ref[pl.ds(0, N), :])


_out = pl.pallas_call(
    _out_body,
    out_shape=jax.ShapeDtypeStruct((N, D), jnp.float32),
)


def kernel(x, edge_index, W, b):
    e2d = edge_index.astype(jnp.int32).reshape(NC * NCHUNK, CH)
    ones = jnp.ones((CH,), jnp.float32)
    zeros1 = jnp.zeros((CH,), jnp.float32)
    zeros2 = jnp.zeros((CH, HD), jnp.float32)

    degp = _deg_kernel(e2d, ones, zeros1)
    degp2 = degp.reshape(NC * N_PAD, 1)
    g0, g1 = _proj(x, W, b.reshape(1, D), degp2)
    acc = _scat_kernel(e2d, g0, g1, zeros2)
    return _out(g0, g1, acc, degp2)
